# baseline (speedup 1.0000x reference)
import jax
import jax.numpy as jnp
from jax.experimental import pallas as pl
from jax.experimental.pallas import tpu as pltpu


def _gate_act(gates, c, G):
    sif = jax.nn.sigmoid(gates[:, :2 * G])
    i_g = sif[:, :G]
    f_g = sif[:, G:]
    g_g = jnp.tanh(gates[:, 2 * G:3 * G])
    o_g = jax.nn.sigmoid(gates[:, 3 * G:])
    c_new = f_g * c + i_g * g_g
    h_new = o_g * jnp.tanh(c_new)
    return h_new, c_new


def _fused_kernel(x_ref, wih0_ref, b0_ref, whh0_ref, wih1_ref, whh1_ref,
                  b1_ref, wmid_ref, bmid_ref, wfc_ref, bfc_ref, out_ref,
                  xg_ref):
    B, T, I = x_ref.shape
    Q, TL, _, G4 = xg_ref.shape
    G = G4 // 4

    wih0 = wih0_ref[...]
    b0 = b0_ref[...]

    def proj_chunk(q):
        xt = jnp.transpose(x_ref[:, q * TL:(q + 1) * TL, :], (1, 0, 2))
        rows = xt.reshape(TL * B, I)
        g = jnp.dot(rows, wih0, preferred_element_type=jnp.float32) + b0
        xg_ref[q] = g.reshape(TL, B, G4)

    whh0 = whh0_ref[...]
    wih1 = wih1_ref[...]
    whh1 = whh1_ref[...]
    w1 = jnp.concatenate([wih1, whh1], axis=0)
    b1 = jnp.broadcast_to(b1_ref[...], (B, G4))

    z = jnp.zeros((B, G), jnp.float32)
    h0, c0, h1, c1 = z, z, z, z
    proj_chunk(0)
    proj_chunk(1)
    for t in range(T):
        if t % TL == 0 and 2 + t // TL < Q:
            proj_chunk(2 + t // TL)
        g0 = xg_ref[t // TL, t % TL] + jnp.dot(
            h0, whh0, preferred_element_type=jnp.float32)
        h0, c0 = _gate_act(g0, c0, G)
        g1 = jnp.dot(jnp.concatenate([h0, h1], axis=-1), w1,
                     preferred_element_type=jnp.float32) + b1
        h1, c1 = _gate_act(g1, c1, G)

    r = jnp.maximum(h1, 0.0)
    mid = (jnp.dot(r, wmid_ref[...], preferred_element_type=jnp.float32)
           + bmid_ref[...])
    out_ref[...] = (jnp.dot(mid, wfc_ref[...],
                            preferred_element_type=jnp.float32)
                    + bfc_ref[...])


@jax.jit
def kernel(x, wih0, whh0, b0, wih1, whh1, b1, wmid, bmid, wfc, bfc):
    B, T, I = x.shape
    G = whh0.shape[0]
    G4 = 4 * G
    H = wmid.shape[1]
    O = wfc.shape[1]
    Q, TL = 4, T // 4

    const = lambda i: (0, 0)
    out = pl.pallas_call(
        _fused_kernel,
        out_shape=jax.ShapeDtypeStruct((B, O), jnp.float32),
        grid=(1,),
        in_specs=[
            pl.BlockSpec((B, T, I), lambda i: (0, 0, 0)),
            pl.BlockSpec((I, G4), const),
            pl.BlockSpec((1, G4), const),
            pl.BlockSpec((G, G4), const),
            pl.BlockSpec((G, G4), const),
            pl.BlockSpec((G, G4), const),
            pl.BlockSpec((1, G4), const),
            pl.BlockSpec((G, H), const),
            pl.BlockSpec((1, H), const),
            pl.BlockSpec((H, O), const),
            pl.BlockSpec((1, O), const),
        ],
        out_specs=pl.BlockSpec((B, O), const),
        scratch_shapes=[
            pltpu.VMEM((Q, TL, B, G4), jnp.float32),
        ],
        compiler_params=pltpu.CompilerParams(
            dimension_semantics=("arbitrary",)),
    )(x, wih0, b0, whh0, wih1, whh1, b1, wmid, bmid, wfc, bfc)
    return out


# speedup vs baseline: 1.0744x; 1.0744x over previous
import jax
import jax.numpy as jnp
from jax.experimental import pallas as pl
from jax.experimental.pallas import tpu as pltpu


def _gate_act(gates, c, G):
    sif = jax.nn.sigmoid(gates[:, :2 * G])
    i_g = sif[:, :G]
    f_g = sif[:, G:]
    g_g = jnp.tanh(gates[:, 2 * G:3 * G])
    o_g = jax.nn.sigmoid(gates[:, 3 * G:])
    c_new = f_g * c + i_g * g_g
    h_new = o_g * jnp.tanh(c_new)
    return h_new, c_new


def _fused_kernel(x_ref, wih0_ref, b0_ref, whh0_ref, wih1_ref, whh1_ref,
                  b1_ref, wmid_ref, bmid_ref, wfc_ref, bfc_ref, out_ref,
                  xg_ref):
    B, T, I = x_ref.shape
    Q, TL, _, G4 = xg_ref.shape
    G = G4 // 4

    wih0 = wih0_ref[...]
    b0 = b0_ref[...]

    def proj_chunk(q):
        xt = jnp.transpose(x_ref[:, q * TL:(q + 1) * TL, :], (1, 0, 2))
        rows = xt.reshape(TL * B, I)
        g = jnp.dot(rows, wih0, preferred_element_type=jnp.float32) + b0
        xg_ref[q] = g.reshape(TL, B, G4)

    whh0 = whh0_ref[...]
    wih1 = wih1_ref[...]
    whh1 = whh1_ref[...]
    b1 = jnp.broadcast_to(b1_ref[...], (B, G4))

    z = jnp.zeros((B, G), jnp.float32)
    h0, c0, h1, c1 = z, z, z, z
    proj_chunk(0)
    proj_chunk(1)
    for t in range(T):
        if t % TL == 0 and 2 + t // TL < Q:
            proj_chunk(2 + t // TL)
        g0 = xg_ref[t // TL, t % TL] + jnp.dot(
            h0, whh0, preferred_element_type=jnp.float32)
        h0, c0 = _gate_act(g0, c0, G)
        g1 = (jnp.dot(h0, wih1, preferred_element_type=jnp.float32)
              + jnp.dot(h1, whh1, preferred_element_type=jnp.float32) + b1)
        h1, c1 = _gate_act(g1, c1, G)

    r = jnp.maximum(h1, 0.0)
    mid = (jnp.dot(r, wmid_ref[...], preferred_element_type=jnp.float32)
           + bmid_ref[...])
    out_ref[...] = (jnp.dot(mid, wfc_ref[...],
                            preferred_element_type=jnp.float32)
                    + bfc_ref[...])


@jax.jit
def kernel(x, wih0, whh0, b0, wih1, whh1, b1, wmid, bmid, wfc, bfc):
    B, T, I = x.shape
    G = whh0.shape[0]
    G4 = 4 * G
    H = wmid.shape[1]
    O = wfc.shape[1]
    Q, TL = 4, T // 4

    const = lambda i: (0, 0)
    out = pl.pallas_call(
        _fused_kernel,
        out_shape=jax.ShapeDtypeStruct((B, O), jnp.float32),
        grid=(1,),
        in_specs=[
            pl.BlockSpec((B, T, I), lambda i: (0, 0, 0)),
            pl.BlockSpec((I, G4), const),
            pl.BlockSpec((1, G4), const),
            pl.BlockSpec((G, G4), const),
            pl.BlockSpec((G, G4), const),
            pl.BlockSpec((G, G4), const),
            pl.BlockSpec((1, G4), const),
            pl.BlockSpec((G, H), const),
            pl.BlockSpec((1, H), const),
            pl.BlockSpec((H, O), const),
            pl.BlockSpec((1, O), const),
        ],
        out_specs=pl.BlockSpec((B, O), const),
        scratch_shapes=[
            pltpu.VMEM((Q, TL, B, G4), jnp.float32),
        ],
        compiler_params=pltpu.CompilerParams(
            dimension_semantics=("arbitrary",)),
    )(x, wih0, b0, whh0, wih1, whh1, b1, wmid, bmid, wfc, bfc)
    return out


# speedup vs baseline: 1.1144x; 1.0372x over previous
import jax
import jax.numpy as jnp
from jax.experimental import pallas as pl
from jax.experimental.pallas import tpu as pltpu


def _gate_act(gates, c, G):
    sif = jax.nn.sigmoid(gates[:, :2 * G])
    i_g = sif[:, :G]
    f_g = sif[:, G:]
    g_g = jnp.tanh(gates[:, 2 * G:3 * G])
    o_g = jax.nn.sigmoid(gates[:, 3 * G:])
    c_new = f_g * c + i_g * g_g
    h_new = o_g * jnp.tanh(c_new)
    return h_new, c_new


def _fused_kernel(x_ref, wih0_ref, b0_ref, whh0_ref, wih1_ref, whh1_ref,
                  b1_ref, wmid_ref, bmid_ref, wfc_ref, bfc_ref, out_ref,
                  xg_ref, h0s_ref):
    B, T, I = x_ref.shape
    Q, TL, _, G4 = xg_ref.shape
    G = G4 // 4

    wih0 = wih0_ref[...]
    b0 = b0_ref[...]

    def proj_chunk(q):
        xt = jnp.transpose(x_ref[:, q * TL:(q + 1) * TL, :], (1, 0, 2))
        rows = xt.reshape(TL * B, I)
        g = jnp.dot(rows, wih0, preferred_element_type=jnp.float32) + b0
        xg_ref[q] = g.reshape(TL, B, G4)

    whh0 = whh0_ref[...]
    wih1 = wih1_ref[...]
    whh1 = whh1_ref[...]
    b1 = b1_ref[...]

    z = jnp.zeros((B, G), jnp.float32)
    h0, c0, h1, c1 = z, z, z, z
    proj_chunk(0)
    proj_chunk(1)
    for q in range(Q):
        if q + 2 < Q:
            proj_chunk(q + 2)
        for tl in range(TL):
            g0 = xg_ref[q, tl] + jnp.dot(
                h0, whh0, preferred_element_type=jnp.float32)
            h0, c0 = _gate_act(g0, c0, G)
            h0s_ref[q, tl] = h0
        hrows = h0s_ref[q].reshape(TL * B, G)
        u = jnp.dot(hrows, wih1, preferred_element_type=jnp.float32) + b1
        xg_ref[q] = u.reshape(TL, B, G4)
        for tl in range(TL):
            g1 = xg_ref[q, tl] + jnp.dot(
                h1, whh1, preferred_element_type=jnp.float32)
            h1, c1 = _gate_act(g1, c1, G)

    r = jnp.maximum(h1, 0.0)
    mid = (jnp.dot(r, wmid_ref[...], preferred_element_type=jnp.float32)
           + bmid_ref[...])
    out_ref[...] = (jnp.dot(mid, wfc_ref[...],
                            preferred_element_type=jnp.float32)
                    + bfc_ref[...])


@jax.jit
def kernel(x, wih0, whh0, b0, wih1, whh1, b1, wmid, bmid, wfc, bfc):
    B, T, I = x.shape
    G = whh0.shape[0]
    G4 = 4 * G
    H = wmid.shape[1]
    O = wfc.shape[1]
    Q, TL = 4, T // 4

    const = lambda i: (0, 0)
    out = pl.pallas_call(
        _fused_kernel,
        out_shape=jax.ShapeDtypeStruct((B, O), jnp.float32),
        grid=(1,),
        in_specs=[
            pl.BlockSpec((B, T, I), lambda i: (0, 0, 0)),
            pl.BlockSpec((I, G4), const),
            pl.BlockSpec((1, G4), const),
            pl.BlockSpec((G, G4), const),
            pl.BlockSpec((G, G4), const),
            pl.BlockSpec((G, G4), const),
            pl.BlockSpec((1, G4), const),
            pl.BlockSpec((G, H), const),
            pl.BlockSpec((1, H), const),
            pl.BlockSpec((H, O), const),
            pl.BlockSpec((1, O), const),
        ],
        out_specs=pl.BlockSpec((B, O), const),
        scratch_shapes=[
            pltpu.VMEM((Q, TL, B, G4), jnp.float32),
            pltpu.VMEM((Q, TL, B, G), jnp.float32),
        ],
        compiler_params=pltpu.CompilerParams(
            dimension_semantics=("arbitrary",)),
    )(x, wih0, b0, whh0, wih1, whh1, b1, wmid, bmid, wfc, bfc)
    return out


# speedup vs baseline: 1.2102x; 1.0860x over previous
import jax
import jax.numpy as jnp
from jax.experimental import pallas as pl
from jax.experimental.pallas import tpu as pltpu


def _gate_act(gates, c, G):
    sif = jax.nn.sigmoid(gates[:, :2 * G])
    i_g = sif[:, :G]
    f_g = sif[:, G:]
    g_g = jnp.tanh(gates[:, 2 * G:3 * G])
    o_g = jax.nn.sigmoid(gates[:, 3 * G:])
    c_new = f_g * c + i_g * g_g
    h_new = o_g * jnp.tanh(c_new)
    return h_new, c_new


def _fused_kernel(x_ref, wih0_ref, b0_ref, whh0_ref, wih1_ref, whh1_ref,
                  b1_ref, wmid_ref, bmid_ref, wfc_ref, bfc_ref, out_ref,
                  xg_ref, h0s_ref):
    B, T, I = x_ref.shape
    Q, TL, _, G4 = xg_ref.shape
    G = G4 // 4

    wih0 = wih0_ref[...]
    b0 = b0_ref[...]

    def proj_chunk(q):
        xt = jnp.transpose(x_ref[:, q * TL:(q + 1) * TL, :], (1, 0, 2))
        rows = xt.reshape(TL * B, I)
        g = jnp.dot(rows, wih0, preferred_element_type=jnp.float32) + b0
        xg_ref[q] = g.reshape(TL, B, G4)

    whh0 = whh0_ref[...]
    wih1 = wih1_ref[...]
    whh1 = whh1_ref[...]
    b1 = b1_ref[...]

    z = jnp.zeros((B, G), jnp.float32)
    h0, c0, h1, c1 = z, z, z, z
    def l0_step(q, tl, h0, c0):
        g0 = xg_ref[q, tl] + jnp.dot(
            h0, whh0, preferred_element_type=jnp.float32)
        h0, c0 = _gate_act(g0, c0, G)
        h0s_ref[q, tl] = h0
        return h0, c0

    def l1_step(q, tl, h1, c1):
        g1 = xg_ref[q, tl] + jnp.dot(
            h1, whh1, preferred_element_type=jnp.float32)
        return _gate_act(g1, c1, G)

    def u_chunk(q):
        hrows = h0s_ref[q].reshape(TL * B, G)
        u = jnp.dot(hrows, wih1, preferred_element_type=jnp.float32) + b1
        xg_ref[q] = u.reshape(TL, B, G4)

    proj_chunk(0)
    proj_chunk(1)
    for tl in range(TL):
        h0, c0 = l0_step(0, tl, h0, c0)
    u_chunk(0)
    for q in range(1, Q):
        if q + 1 < Q:
            proj_chunk(q + 1)
        for tl in range(TL):
            h0, c0 = l0_step(q, tl, h0, c0)
            h1, c1 = l1_step(q - 1, tl, h1, c1)
        u_chunk(q)
    for tl in range(TL):
        h1, c1 = l1_step(Q - 1, tl, h1, c1)

    r = jnp.maximum(h1, 0.0)
    mid = (jnp.dot(r, wmid_ref[...], preferred_element_type=jnp.float32)
           + bmid_ref[...])
    out_ref[...] = (jnp.dot(mid, wfc_ref[...],
                            preferred_element_type=jnp.float32)
                    + bfc_ref[...])


@jax.jit
def kernel(x, wih0, whh0, b0, wih1, whh1, b1, wmid, bmid, wfc, bfc):
    B, T, I = x.shape
    G = whh0.shape[0]
    G4 = 4 * G
    H = wmid.shape[1]
    O = wfc.shape[1]
    Q, TL = 4, T // 4

    const = lambda i: (0, 0)
    out = pl.pallas_call(
        _fused_kernel,
        out_shape=jax.ShapeDtypeStruct((B, O), jnp.float32),
        grid=(1,),
        in_specs=[
            pl.BlockSpec((B, T, I), lambda i: (0, 0, 0)),
            pl.BlockSpec((I, G4), const),
            pl.BlockSpec((1, G4), const),
            pl.BlockSpec((G, G4), const),
            pl.BlockSpec((G, G4), const),
            pl.BlockSpec((G, G4), const),
            pl.BlockSpec((1, G4), const),
            pl.BlockSpec((G, H), const),
            pl.BlockSpec((1, H), const),
            pl.BlockSpec((H, O), const),
            pl.BlockSpec((1, O), const),
        ],
        out_specs=pl.BlockSpec((B, O), const),
        scratch_shapes=[
            pltpu.VMEM((Q, TL, B, G4), jnp.float32),
            pltpu.VMEM((Q, TL, B, G), jnp.float32),
        ],
        compiler_params=pltpu.CompilerParams(
            dimension_semantics=("arbitrary",)),
    )(x, wih0, b0, whh0, wih1, whh1, b1, wmid, bmid, wfc, bfc)
    return out


# speedup vs baseline: 1.2451x; 1.0289x over previous
import jax
import jax.numpy as jnp
from jax.experimental import pallas as pl
from jax.experimental.pallas import tpu as pltpu


def _gate_act(gates, c, G):
    sif = jax.nn.sigmoid(gates[:, :2 * G])
    i_g = sif[:, :G]
    f_g = sif[:, G:]
    g_g = jnp.tanh(gates[:, 2 * G:3 * G])
    o_g = jax.nn.sigmoid(gates[:, 3 * G:])
    c_new = f_g * c + i_g * g_g
    h_new = o_g * jnp.tanh(c_new)
    return h_new, c_new


def _fused_kernel(x_ref, wih0_ref, b0_ref, whh0_ref, wih1_ref, whh1_ref,
                  b1_ref, wmid_ref, bmid_ref, wfc_ref, bfc_ref, out_ref,
                  xg_ref, h0s_ref, xt_ref):
    B, T, I = x_ref.shape
    Q, TL, _, G4 = xg_ref.shape
    G = G4 // 4

    wih0 = wih0_ref[...]
    b0 = b0_ref[...]

    def proj_chunk(q):
        xt = jnp.transpose(x_ref[:, q * TL:(q + 1) * TL, :], (1, 0, 2))
        rows = xt.reshape(TL * B, I)
        g = jnp.dot(rows, wih0, preferred_element_type=jnp.float32) + b0
        xg_ref[q] = g.reshape(TL, B, G4)

    whh0 = whh0_ref[...]
    wih1 = wih1_ref[...]
    whh1 = whh1_ref[...]
    b1 = b1_ref[...]

    z = jnp.zeros((B, G), jnp.float32)
    h0, c0, h1, c1 = z, z, z, z
    def l0_step(q, tl, h0, c0):
        g0 = xg_ref[q, tl] + jnp.dot(
            h0, whh0, preferred_element_type=jnp.float32)
        h0, c0 = _gate_act(g0, c0, G)
        h0s_ref[q, tl] = h0
        return h0, c0

    def l1_step(q, tl, h1, c1):
        g1 = xg_ref[q, tl] + jnp.dot(
            h1, whh1, preferred_element_type=jnp.float32)
        return _gate_act(g1, c1, G)

    def u_chunk(q):
        hrows = h0s_ref[q].reshape(TL * B, G)
        u = jnp.dot(hrows, wih1, preferred_element_type=jnp.float32) + b1
        xg_ref[q] = u.reshape(TL, B, G4)

    def xt_store(q):
        xt = jnp.transpose(x_ref[:, q * TL:(q + 1) * TL, :], (1, 0, 2))
        xt_ref[...] = xt.reshape(TL * B, I)

    NS = G4 // 256

    def proj_sub(q, j):
        cols = slice(j * 256, (j + 1) * 256)
        g = jnp.dot(xt_ref[...], wih0[:, cols],
                    preferred_element_type=jnp.float32) + b0[:, cols]
        xg_ref[q, :, :, cols] = g.reshape(TL, B, 256)

    proj_chunk(0)
    xt_store(1)
    for tl in range(TL):
        h0, c0 = l0_step(0, tl, h0, c0)
        proj_sub(1, tl)
    u_chunk(0)
    for q in range(1, Q):
        if q + 1 < Q:
            xt_store(q + 1)
        for tl in range(TL):
            h0, c0 = l0_step(q, tl, h0, c0)
            h1, c1 = l1_step(q - 1, tl, h1, c1)
            if q + 1 < Q:
                proj_sub(q + 1, tl)
        u_chunk(q)
    for tl in range(TL):
        h1, c1 = l1_step(Q - 1, tl, h1, c1)

    r = jnp.maximum(h1, 0.0)
    mid = (jnp.dot(r, wmid_ref[...], preferred_element_type=jnp.float32)
           + bmid_ref[...])
    out_ref[...] = (jnp.dot(mid, wfc_ref[...],
                            preferred_element_type=jnp.float32)
                    + bfc_ref[...])


@jax.jit
def kernel(x, wih0, whh0, b0, wih1, whh1, b1, wmid, bmid, wfc, bfc):
    B, T, I = x.shape
    G = whh0.shape[0]
    G4 = 4 * G
    H = wmid.shape[1]
    O = wfc.shape[1]
    Q, TL = 4, T // 4

    const = lambda i: (0, 0)
    out = pl.pallas_call(
        _fused_kernel,
        out_shape=jax.ShapeDtypeStruct((B, O), jnp.float32),
        grid=(1,),
        in_specs=[
            pl.BlockSpec((B, T, I), lambda i: (0, 0, 0)),
            pl.BlockSpec((I, G4), const),
            pl.BlockSpec((1, G4), const),
            pl.BlockSpec((G, G4), const),
            pl.BlockSpec((G, G4), const),
            pl.BlockSpec((G, G4), const),
            pl.BlockSpec((1, G4), const),
            pl.BlockSpec((G, H), const),
            pl.BlockSpec((1, H), const),
            pl.BlockSpec((H, O), const),
            pl.BlockSpec((1, O), const),
        ],
        out_specs=pl.BlockSpec((B, O), const),
        scratch_shapes=[
            pltpu.VMEM((Q, TL, B, G4), jnp.float32),
            pltpu.VMEM((Q, TL, B, G), jnp.float32),
            pltpu.VMEM((T // 4 * B, I), jnp.float32),
        ],
        compiler_params=pltpu.CompilerParams(
            dimension_semantics=("arbitrary",)),
    )(x, wih0, b0, whh0, wih1, whh1, b1, wmid, bmid, wfc, bfc)
    return out
